# baseline (speedup 1.0000x reference)
import jax
import jax.numpy as jnp
from jax import lax
from jax.experimental import pallas as pl
from jax.experimental.pallas import tpu as pltpu
from jax.experimental.pallas import tpu_sc as plsc

_N = 2 * 4096 * 2048
_COLS = 2048
_ROWS = _N // _COLS
_K_SC = 12
_NW = 32
_BLK = 512

_N_SC = _K_SC * (_N // 32)
_N_TC = _N - _N_SC
_R_TC = _N_TC // _COLS
_R_SC = _N_SC // _COLS
_TC_GRID = _R_TC // _BLK

_CR = 8
_C = _CR * _COLS
_CW = _C // 4
_SPAN_R = _R_SC // _NW
_NCH = _SPAN_R // _CR


def _tc_body(x_ref, y_ref, w_ref, o_ref):
    i = pl.program_id(0)

    @pl.when(i == 0)
    def _init():
        o_ref[0] = 0.0
        o_ref[1] = 0.0
        o_ref[2] = 0.0
        o_ref[3] = 0.0

    d = jnp.abs(x_ref[...] - y_ref[...])
    t = w_ref[...].astype(jnp.int32)
    w0 = (t & 1).astype(jnp.float32)
    w1 = ((t >> 1) & 1).astype(jnp.float32)
    o_ref[0] += jnp.sum(d * w0)
    o_ref[1] += jnp.sum(w0)
    o_ref[2] += jnp.sum(d * w1)
    o_ref[3] += jnp.sum(w1)


def _swar_bytes_sum(a):
    return ((a & 255) + ((a >> 8) & 255) + ((a >> 16) & 255) + ((a >> 24) & 255))


def _sc_body(x_hbm, y_hbm, w_hbm, out_hbm, x_v, y_v, w_v, o_v, sem_a, sem_b):
    wid = lax.axis_index("s") * 2 + lax.axis_index("c")
    row0 = _R_TC + wid * _SPAN_R
    word0 = wid * (_SPAN_R * _COLS // 4)

    iotav = lax.iota(jnp.int32, 16)
    idx4 = [(iotav >> 2) + 4 * k for k in range(4)]
    sh0 = (iotav & 3) * 8
    sh1 = sh0 + 1

    def start(ci, par, sem):
        r = row0 + ci * _CR
        pltpu.async_copy(x_hbm.at[pl.ds(r, _CR)], x_v.at[par], sem)
        pltpu.async_copy(y_hbm.at[pl.ds(r, _CR)], y_v.at[par], sem)
        pltpu.async_copy(w_hbm.at[pl.ds(word0 + ci * _CW, _CW)], w_v.at[par], sem)

    def drain(par, sem):
        pltpu.make_async_copy(x_hbm.at[pl.ds(0, _CR)], x_v.at[par], sem).wait()
        pltpu.make_async_copy(y_hbm.at[pl.ds(0, _CR)], y_v.at[par], sem).wait()
        pltpu.make_async_copy(w_hbm.at[pl.ds(0, _CW)], w_v.at[par], sem).wait()

    def comp(par, carry):
        s0i, s1i, c0i, c1i = carry

        def gbody(g, gc):
            s0, s1, b0c, b1c = gc
            row = g >> 5
            col0 = (g & 31) * 64
            words = w_v[par, pl.ds(g * 16, 16)]
            b0c = b0c + (words & 0x01010101)
            b1c = b1c + ((words >> 1) & 0x01010101)
            for k in range(4):
                xv = x_v[par, row, pl.ds(col0 + k * 16, 16)]
                yv = y_v[par, row, pl.ds(col0 + k * 16, 16)]
                d = jnp.abs(xv - yv)
                g16 = words.at[idx4[k]].get(mode="promise_in_bounds")
                w0 = ((g16 >> sh0) & 1).astype(jnp.float32)
                w1 = ((g16 >> sh1) & 1).astype(jnp.float32)
                s0 = s0 + d * w0
                s1 = s1 + d * w1
            return (s0, s1, b0c, b1c)

        zero = jnp.zeros((16,), jnp.int32)
        half = _C // 128
        for h in range(2):
            s0i, s1i, b0c, b1c = lax.fori_loop(
                h * half, (h + 1) * half, gbody, (s0i, s1i, zero, zero))
            c0i = c0i + _swar_bytes_sum(b0c)
            c1i = c1i + _swar_bytes_sum(b1c)
        return (s0i, s1i, c0i, c1i)

    start(0, 0, sem_a)
    start(1, 1, sem_b)

    zf = jnp.zeros((16,), jnp.float32)
    zi = jnp.zeros((16,), jnp.int32)

    def sbody(si, carry):
        c0 = 2 * si
        drain(0, sem_a)
        carry = comp(0, carry)

        @pl.when(c0 + 2 < _NCH)
        def _():
            start(c0 + 2, 0, sem_a)

        drain(1, sem_b)
        carry = comp(1, carry)

        @pl.when(c0 + 3 < _NCH)
        def _():
            start(c0 + 3, 1, sem_b)

        return carry

    s0v, s1v, c0v, c1v = lax.fori_loop(0, _NCH // 2, sbody, (zf, zf, zi, zi))

    o_v[0, :] = s0v
    o_v[1, :] = c0v.astype(jnp.float32)
    o_v[2, :] = s1v
    o_v[3, :] = c1v.astype(jnp.float32)
    pltpu.sync_copy(o_v, out_hbm.at[wid])


def _make_sc_call(interpret=False):
    mesh = plsc.VectorSubcoreMesh(core_axis_name="c", subcore_axis_name="s")
    return pl.kernel(
        _sc_body,
        out_type=jax.ShapeDtypeStruct((_NW, 4, 16), jnp.float32),
        mesh=mesh,
        scratch_types=[
            pltpu.VMEM((2, _CR, _COLS), jnp.float32),
            pltpu.VMEM((2, _CR, _COLS), jnp.float32),
            pltpu.VMEM((2, _CW), jnp.int32),
            pltpu.VMEM((4, 16), jnp.float32),
            pltpu.SemaphoreType.DMA,
            pltpu.SemaphoreType.DMA,
        ],
        compiler_params=pltpu.CompilerParams(use_tc_tiling_on_sc=True),
        interpret=interpret,
    )


def kernel(input, output, mask0, mask1):
    x = input.reshape(_ROWS, _COLS)
    y = output.reshape(_ROWS, _COLS)
    m0 = mask0.reshape(_ROWS, _COLS)
    m1 = mask1.reshape(_ROWS, _COLS)

    w01 = ((~m0[:_R_TC]).astype(jnp.int8)
           | ((~m1[:_R_TC]).astype(jnp.int8) << 1))

    mt0 = mask0.reshape(-1)[_N_TC:].reshape(-1, 4)
    mt1 = mask1.reshape(-1)[_N_TC:].reshape(-1, 4)
    bsh = ((~mt0).astype(jnp.int32) | ((~mt1).astype(jnp.int32) << 1)) << (
        jnp.arange(4, dtype=jnp.int32) * 8)
    wsc = jnp.sum(bsh, axis=1, dtype=jnp.int32)

    sc_part = _make_sc_call()(x, y, wsc)

    spec = pl.BlockSpec((_BLK, _COLS), lambda i: (i, 0))
    tc_part = pl.pallas_call(
        _tc_body,
        grid=(_TC_GRID,),
        in_specs=[spec, spec, spec],
        out_specs=pl.BlockSpec(memory_space=pltpu.SMEM),
        out_shape=jax.ShapeDtypeStruct((4,), jnp.float32),
        compiler_params=pltpu.CompilerParams(
            dimension_semantics=("arbitrary",),
        ),
    )(x, y, w01)

    parts = tc_part + sc_part.sum(axis=(0, 2))
    return parts[0] / parts[1] + parts[2] / parts[3]


# speedup vs baseline: 33.9711x; 33.9711x over previous
import jax
import jax.numpy as jnp
from jax import lax
from jax.experimental import pallas as pl
from jax.experimental.pallas import tpu as pltpu
from jax.experimental.pallas import tpu_sc as plsc

_N = 2 * 4096 * 2048
_COLS = 2048
_ROWS = _N // _COLS
_K_SC = 10
_NW = 32
_BLK = 512

_N_SC = _K_SC * (_N // 32)
_N_TC = _N - _N_SC
_R_TC = _N_TC // _COLS
_R_SC = _N_SC // _COLS
_TC_GRID = _R_TC // _BLK

_CR = 8
_C = _CR * _COLS
_SPAN_R = _R_SC // _NW
_NCH = _SPAN_R // _CR
_UNROLL = 4


def _tc_body(x_ref, y_ref, w_ref, o_ref):
    i = pl.program_id(0)

    @pl.when(i == 0)
    def _init():
        o_ref[0] = 0.0
        o_ref[1] = 0.0
        o_ref[2] = 0.0
        o_ref[3] = 0.0

    d = jnp.abs(x_ref[...] - y_ref[...])
    t = w_ref[...].astype(jnp.int32)
    w0 = (t & 1).astype(jnp.float32)
    w1 = ((t >> 1) & 1).astype(jnp.float32)
    o_ref[0] += jnp.sum(d * w0)
    o_ref[1] += jnp.sum(w0)
    o_ref[2] += jnp.sum(d * w1)
    o_ref[3] += jnp.sum(w1)


def _sc_body(x_hbm, y_hbm, w_hbm, out_hbm, x_v, y_v, w_v, o_v, sem_a, sem_b):
    wid = lax.axis_index("s") * 2 + lax.axis_index("c")
    row0 = _R_TC + wid * _SPAN_R
    elt0 = wid * (_SPAN_R * _COLS)

    def start(ci, par, sem):
        r = row0 + ci * _CR
        pltpu.async_copy(x_hbm.at[pl.ds(r, _CR)], x_v.at[par], sem)
        pltpu.async_copy(y_hbm.at[pl.ds(r, _CR)], y_v.at[par], sem)
        pltpu.async_copy(w_hbm.at[pl.ds(elt0 + ci * _C, _C)], w_v.at[par], sem)

    def drain(par, sem):
        pltpu.make_async_copy(x_hbm.at[pl.ds(0, _CR)], x_v.at[par], sem).wait()
        pltpu.make_async_copy(y_hbm.at[pl.ds(0, _CR)], y_v.at[par], sem).wait()
        pltpu.make_async_copy(w_hbm.at[pl.ds(0, _C)], w_v.at[par], sem).wait()

    def comp(par, carry):
        def gbody(g, gc):
            s0, s1, c0, c1 = gc
            for u in range(_UNROLL):
                v = g * _UNROLL + u
                row = v >> 7
                col = (v & 127) * 16
                xv = x_v[par, row, pl.ds(col, 16)]
                yv = y_v[par, row, pl.ds(col, 16)]
                t = w_v[par, pl.ds(v * 16, 16)]
                w0i = t & 1
                w1i = t >> 1
                d = jnp.abs(xv - yv)
                s0 = s0 + d * w0i.astype(jnp.float32)
                s1 = s1 + d * w1i.astype(jnp.float32)
                c0 = c0 + w0i
                c1 = c1 + w1i
            return (s0, s1, c0, c1)

        return lax.fori_loop(0, _C // (16 * _UNROLL), gbody, carry)

    start(0, 0, sem_a)
    start(1, 1, sem_b)

    zf = jnp.zeros((16,), jnp.float32)
    zi = jnp.zeros((16,), jnp.int32)

    def sbody(si, carry):
        c0 = 2 * si
        drain(0, sem_a)
        carry = comp(0, carry)

        @pl.when(c0 + 2 < _NCH)
        def _():
            start(c0 + 2, 0, sem_a)

        drain(1, sem_b)
        carry = comp(1, carry)

        @pl.when(c0 + 3 < _NCH)
        def _():
            start(c0 + 3, 1, sem_b)

        return carry

    s0v, s1v, c0v, c1v = lax.fori_loop(0, _NCH // 2, sbody, (zf, zf, zi, zi))

    o_v[0, :] = s0v
    o_v[1, :] = c0v.astype(jnp.float32)
    o_v[2, :] = s1v
    o_v[3, :] = c1v.astype(jnp.float32)
    pltpu.sync_copy(o_v, out_hbm.at[wid])


def _make_sc_call(interpret=False):
    mesh = plsc.VectorSubcoreMesh(core_axis_name="c", subcore_axis_name="s")
    return pl.kernel(
        _sc_body,
        out_type=jax.ShapeDtypeStruct((_NW, 4, 16), jnp.float32),
        mesh=mesh,
        scratch_types=[
            pltpu.VMEM((2, _CR, _COLS), jnp.float32),
            pltpu.VMEM((2, _CR, _COLS), jnp.float32),
            pltpu.VMEM((2, _C), jnp.int32),
            pltpu.VMEM((4, 16), jnp.float32),
            pltpu.SemaphoreType.DMA,
            pltpu.SemaphoreType.DMA,
        ],
        compiler_params=pltpu.CompilerParams(use_tc_tiling_on_sc=True),
        interpret=interpret,
    )


def kernel(input, output, mask0, mask1):
    x = input.reshape(_ROWS, _COLS)
    y = output.reshape(_ROWS, _COLS)
    m0 = mask0.reshape(_ROWS, _COLS)
    m1 = mask1.reshape(_ROWS, _COLS)

    w01 = ((~m0[:_R_TC]).astype(jnp.int8)
           | ((~m1[:_R_TC]).astype(jnp.int8) << 1))

    mt0 = mask0.reshape(-1)[_N_TC:]
    mt1 = mask1.reshape(-1)[_N_TC:]
    wsc = (~mt0).astype(jnp.int32) | ((~mt1).astype(jnp.int32) << 1)

    sc_part = _make_sc_call()(x, y, wsc)

    spec = pl.BlockSpec((_BLK, _COLS), lambda i: (i, 0))
    tc_part = pl.pallas_call(
        _tc_body,
        grid=(_TC_GRID,),
        in_specs=[spec, spec, spec],
        out_specs=pl.BlockSpec(memory_space=pltpu.SMEM),
        out_shape=jax.ShapeDtypeStruct((4,), jnp.float32),
        compiler_params=pltpu.CompilerParams(
            dimension_semantics=("arbitrary",),
        ),
    )(x, y, w01)

    parts = tc_part + sc_part.sum(axis=(0, 2))
    return parts[0] / parts[1] + parts[2] / parts[3]


# speedup vs baseline: 81.1838x; 2.3898x over previous
import jax
import jax.numpy as jnp
from jax.experimental import pallas as pl
from jax.experimental.pallas import tpu as pltpu

_BATCH = 2
_ROWS = 4096
_COLS = 2048
_BLK = 512
_GRID = _BATCH * _ROWS // _BLK


def _body(x_ref, y_ref, w_ref, o_ref, acc_ref):
    i = pl.program_id(0)

    @pl.when(i == 0)
    def _init():
        acc_ref[0] = 0.0
        acc_ref[1] = 0.0
        acc_ref[2] = 0.0
        acc_ref[3] = 0.0

    d = jnp.abs(x_ref[...] - y_ref[...])
    t = w_ref[...].astype(jnp.int32)
    w0 = (t & 1).astype(jnp.float32)
    w1 = (t >> 1).astype(jnp.float32)
    acc_ref[0] += jnp.sum(d * w0)
    acc_ref[1] += jnp.sum(w0)
    acc_ref[2] += jnp.sum(d * w1)
    acc_ref[3] += jnp.sum(w1)

    @pl.when(i == _GRID - 1)
    def _fin():
        o_ref[0] = acc_ref[0] / acc_ref[1] + acc_ref[2] / acc_ref[3]


def kernel(input, output, mask0, mask1):
    w01 = (~mask0).astype(jnp.int8) + ((~mask1).astype(jnp.int8) << 1)

    blocks_per_batch = _ROWS // _BLK
    spec = pl.BlockSpec(
        (1, _BLK, _COLS),
        lambda i: (i // blocks_per_batch, i % blocks_per_batch, 0),
    )
    out = pl.pallas_call(
        _body,
        grid=(_GRID,),
        in_specs=[spec, spec, spec],
        out_specs=pl.BlockSpec(memory_space=pltpu.SMEM),
        out_shape=jax.ShapeDtypeStruct((1,), jnp.float32),
        scratch_shapes=[pltpu.SMEM((4,), jnp.float32)],
        compiler_params=pltpu.CompilerParams(
            dimension_semantics=("arbitrary",),
        ),
    )(input, output, w01)
    return out[0]
